# Initial kernel scaffold; baseline (speedup 1.0000x reference)
import jax
import jax.numpy as jnp
from jax.experimental import pallas as pl


def kernel(x, lut):
    raise NotImplementedError("write your pallas kernel here")


# speedup vs baseline: 1.5416x; 1.5416x over previous
import functools
import math

import jax
import jax.numpy as jnp
from jax import lax
from jax.experimental import pallas as pl
from jax.experimental.pallas import tpu as pltpu
from jax.experimental.pallas import tpu_sc as plsc

_D = 512
_SCALE = math.sqrt(_D)
_NC, _NS = 2, 16
_NW = _NC * _NS
_CHUNK = 64
_NBUF = 3
_LANES = 16


def _make_scaled_gather(n, d):
    per_w = n // _NW
    n_chunks = per_w // _CHUNK
    mesh = plsc.VectorSubcoreMesh(
        core_axis_name="c", subcore_axis_name="s",
        num_cores=_NC, num_subcores=_NS)

    @functools.partial(
        pl.kernel,
        out_type=jax.ShapeDtypeStruct((n, d), jnp.float32),
        mesh=mesh,
        scratch_types=[
            pltpu.VMEM((per_w,), jnp.int32),
            *[pltpu.VMEM((_CHUNK, d), jnp.float32) for _ in range(_NBUF)],
            *[pltpu.SemaphoreType.DMA for _ in range(2 * _NBUF)],
        ],
    )
    def emb(x_hbm, lut_hbm, out_hbm, idx_v, *rest):
        rows = rest[:_NBUF]
        in_sems = rest[_NBUF:2 * _NBUF]
        out_sems = rest[2 * _NBUF:]
        wid = lax.axis_index("s") * _NC + lax.axis_index("c")
        base = wid * per_w
        pltpu.sync_copy(x_hbm.at[pl.ds(base, per_w)], idx_v)

        def start_gather(c):
            b = c % _NBUF
            return pltpu.async_copy(
                lut_hbm.at[idx_v.at[pl.ds(c * _CHUNK, _CHUNK)]],
                rows[b], in_sems[b])

        gathers = {0: start_gather(0)}
        stores = {}
        for c in range(n_chunks):
            b = c % _NBUF
            nxt = c + 1
            if nxt < n_chunks:
                if nxt - _NBUF in stores:
                    stores.pop(nxt - _NBUF).wait()
                gathers[nxt] = start_gather(nxt)
            gathers.pop(c).wait()

            def row_body(r, acc, _b=b):
                for j in range(d // _LANES):
                    sl = (r, pl.ds(j * _LANES, _LANES))
                    rows[_b][sl] = rows[_b][sl] * _SCALE
                return acc
            lax.fori_loop(0, _CHUNK, row_body, 0)

            stores[c] = pltpu.async_copy(
                rows[b], out_hbm.at[pl.ds(base + c * _CHUNK, _CHUNK)],
                out_sems[b])
        for c in sorted(stores):
            stores.pop(c).wait()

    return emb


def kernel(x, lut):
    b, s = x.shape
    x_flat = x.reshape(-1).astype(jnp.int32)
    out = _make_scaled_gather(x_flat.shape[0], lut.shape[1])(x_flat, lut)
    return out.reshape(b, s, lut.shape[1])
